# baseline (speedup 1.0000x reference)
import functools

import jax
import jax.numpy as jnp
import numpy as np
from jax.experimental import pallas as pl
from jax.experimental.pallas import tpu as pltpu

_B, _C, _T, _H = 1024, 8, 2048, 24
_K = 4
_THRESHOLD = 0.001
_BB = 32
_R = _BB * _C
_TWO_PI_OVER_T = 2.0 * np.pi / _T


def _topk_idx(vals, ti):
    v = jnp.abs(vals)
    idxs = []
    kth = None
    for _ in range(_K):
        m = jnp.max(v, axis=1, keepdims=True)
        am = jnp.min(jnp.where(v == m, ti, _T), axis=1, keepdims=True)
        v = jnp.where(ti == am, -1.0, v)
        idxs.append(am)
        kth = m
    return idxs, kth


def _cos2pi(v):
    r = v - jnp.round(v)
    a = jnp.abs(r)
    flip = a > 0.25
    a = jnp.where(flip, 0.5 - a, a)
    x2 = (a * (2.0 * np.pi)) ** 2
    c = 1.0 + x2 * (-0.5 + x2 * (1.0 / 24 + x2 * (-1.0 / 720 + x2 * (
        1.0 / 40320 + x2 * (-1.0 / 3628800 + x2 * (1.0 / 479001600))))))
    return jnp.where(flip, -c, c)


def _dft_sq_sum(attrib, idxs, tif, use_sin):
    ss = jnp.zeros((attrib.shape[0], 1), jnp.float32)
    for k in range(_K):
        fscaled = idxs[k].astype(jnp.float32) * (1.0 / _T)
        v = fscaled * tif
        if use_sin:
            v = v - 0.25
        b = jnp.sum(attrib * _cos2pi(v), axis=1, keepdims=True)
        ss = ss + b * b
    return ss


def _pass1_kernel(x_ref, er_ref, ei_ref, s_ref,
                  idxi_ref, v4r_ref, v4i_ref, rss_ref, iss_ref):
    attrib = x_ref[...] * s_ref[...]
    ti = jax.lax.broadcasted_iota(jnp.int32, (_R, _T), 1)
    tif = ti.astype(jnp.float32)
    idx_r, v4r = _topk_idx(er_ref[...], ti)
    idx_i, v4i = _topk_idx(ei_ref[...], ti)
    rss_ref[...] = _dft_sq_sum(attrib, idx_r, tif, use_sin=False)
    iss_ref[...] = _dft_sq_sum(attrib, idx_i, tif, use_sin=True)
    v4r_ref[...] = v4r
    v4i_ref[...] = v4i
    idxi_ref[...] = jnp.concatenate(idx_i, axis=1)


def _pass2_kernel(rows_ref, x_ref, idx_ref, s_ref, out_ref):
    del rows_ref
    attrib = x_ref[...] * s_ref[...]
    ti = jax.lax.broadcasted_iota(jnp.int32, (_C, _T), 1)
    idx = idx_ref[...]
    ss = jnp.zeros((_C, 1), jnp.float32)
    for k in range(_K):
        prod = jnp.bitwise_and(idx[:, k:k + 1] * ti, _T - 1)
        ang = prod.astype(jnp.float32) * _TWO_PI_OVER_T
        b = jnp.sum(attrib * jnp.sin(ang), axis=1, keepdims=True)
        ss = ss + b * b
    out_ref[...] = jnp.full((1, 1, 128), jnp.sum(ss), jnp.float32)


def kernel(input, predictions, expl_p_real, expl_p_imag, W):
    del predictions
    s = jnp.sum(W, axis=1).reshape(1, _T)
    x2 = input.reshape(_B * _C, _T)
    er2 = expl_p_real.reshape(_B * _C, _T)
    ei2 = expl_p_imag.reshape(_B * _C, _T)

    nb = _B // _BB
    idxi, v4r, v4i, rss, iss_self = pl.pallas_call(
        _pass1_kernel,
        grid=(nb,),
        compiler_params=pltpu.CompilerParams(
            dimension_semantics=("parallel",)),
        in_specs=[
            pl.BlockSpec((_R, _T), lambda i: (i, 0)),
            pl.BlockSpec((_R, _T), lambda i: (i, 0)),
            pl.BlockSpec((_R, _T), lambda i: (i, 0)),
            pl.BlockSpec((1, _T), lambda i: (0, 0)),
        ],
        out_specs=[
            pl.BlockSpec((_R, _K), lambda i: (i, 0)),
            pl.BlockSpec((_R, 1), lambda i: (i, 0)),
            pl.BlockSpec((_R, 1), lambda i: (i, 0)),
            pl.BlockSpec((_R, 1), lambda i: (i, 0)),
            pl.BlockSpec((_R, 1), lambda i: (i, 0)),
        ],
        out_shape=[
            jax.ShapeDtypeStruct((_B * _C, _K), jnp.int32),
            jax.ShapeDtypeStruct((_B * _C, 1), jnp.float32),
            jax.ShapeDtypeStruct((_B * _C, 1), jnp.float32),
            jax.ShapeDtypeStruct((_B * _C, 1), jnp.float32),
            jax.ShapeDtypeStruct((_B * _C, 1), jnp.float32),
        ],
    )(x2, er2, ei2, s)

    real_invalid = v4r.reshape(_B, _C).min(axis=1) < _THRESHOLD
    imag_invalid = v4i.reshape(_B, _C).min(axis=1) < _THRESHOLD
    kept_real = jnp.logical_not(real_invalid)
    kept_imag = jnp.logical_not(imag_invalid)
    real_count = jnp.sum(kept_real)
    imag_count = jnp.sum(kept_imag)
    ck = float(_C * _K)

    rtot = jnp.sum(rss.reshape(_B, _C).sum(axis=1)
                   * kept_real.astype(jnp.float32))
    rden = jnp.maximum(real_count.astype(jnp.float32), 1.0)
    real_loss = jnp.where(real_count > 0, rtot / (rden * ck) / rden,
                          jnp.float32(0.0))

    def _imag_common(_):
        return jnp.sum(iss_self)

    def _imag_paired(_):
        row_ids = jnp.arange(_B)
        fft_mask = jnp.where(real_count > 0, kept_real,
                             jnp.ones_like(kept_real))
        sym_mask = jnp.logical_xor(fft_mask, imag_invalid)
        sym_sorted = jnp.sort(jnp.where(sym_mask, row_ids, _B))
        kept_sorted = jnp.sort(jnp.where(kept_imag, row_ids, _B))
        rows_fft = jnp.minimum(sym_sorted, _B - 1).astype(jnp.int32)
        rows_idx = jnp.minimum(kept_sorted, _B - 1).astype(jnp.int32)
        idx_pair = idxi.reshape(_B, _C, _K)[rows_idx].reshape(_B * _C, _K)
        pair_valid = (row_ids < imag_count).astype(jnp.float32)
        iss2 = pl.pallas_call(
            _pass2_kernel,
            grid_spec=pltpu.PrefetchScalarGridSpec(
                num_scalar_prefetch=1,
                grid=(_B,),
                in_specs=[
                    pl.BlockSpec((_C, _T), lambda r, rows: (rows[r], 0)),
                    pl.BlockSpec((_C, _K), lambda r, rows: (r, 0)),
                    pl.BlockSpec((1, _T), lambda r, rows: (0, 0)),
                ],
                out_specs=pl.BlockSpec((1, 1, 128), lambda r, rows: (r, 0, 0)),
            ),
            out_shape=jax.ShapeDtypeStruct((_B, 1, 128), jnp.float32),
        )(rows_fft, x2, idx_pair, s)
        return jnp.sum(iss2[:, 0, 0] * pair_valid)

    all_valid = jnp.logical_and(real_count == _B, imag_count == _B)
    itot = jax.lax.cond(all_valid, _imag_common, _imag_paired, None)
    iden = jnp.maximum(imag_count.astype(jnp.float32), 1.0)
    imag_loss = jnp.where(imag_count > 0, itot / (iden * ck) / iden,
                          jnp.float32(0.0))
    return real_loss + imag_loss


# speedup vs baseline: 1.1672x; 1.1672x over previous
import functools

import jax
import jax.numpy as jnp
import numpy as np
from jax.experimental import pallas as pl
from jax.experimental.pallas import tpu as pltpu

_B, _C, _T, _H = 1024, 8, 2048, 24
_K = 4
_THRESHOLD = 0.001
_BB = 32
_R = _BB * _C
_TWO_PI_OVER_T = 2.0 * np.pi / _T


def _topk_idx(vals, ti):
    v = jnp.abs(vals)
    idxs = []
    kth = None
    for _ in range(_K):
        m = jnp.max(v, axis=1, keepdims=True)
        am = jnp.min(jnp.where(v == m, ti, _T), axis=1, keepdims=True)
        v = jnp.where(ti == am, -1.0, v)
        idxs.append(am)
        kth = m
    return idxs, kth


def _cos2pi(v):
    r = v - jnp.round(v)
    u = r * r
    return 0.999999999884617 + u * (-19.739208743087502 + u * (
        64.9393890524762 + u * (-85.45665775339239 + u * (
            60.24212492771494 + u * (-26.404630603916303 + u * (
                7.800022165291011 + u * -1.4529874920845032))))))


def _dft_sq_sum(attrib, idxs, tif, use_sin):
    ss = jnp.zeros((attrib.shape[0], 1), jnp.float32)
    for k in range(_K):
        fscaled = idxs[k].astype(jnp.float32) * (1.0 / _T)
        v = fscaled * tif
        if use_sin:
            v = v - 0.25
        b = jnp.sum(attrib * _cos2pi(v), axis=1, keepdims=True)
        ss = ss + b * b
    return ss


def _pass1_kernel(x_ref, er_ref, ei_ref, s_ref,
                  idxi_ref, v4r_ref, v4i_ref, rss_ref, iss_ref):
    attrib = x_ref[...] * s_ref[...]
    ti = jax.lax.broadcasted_iota(jnp.int32, (_R, _T), 1)
    tif = ti.astype(jnp.float32)
    idx_r, v4r = _topk_idx(er_ref[...], ti)
    idx_i, v4i = _topk_idx(ei_ref[...], ti)
    rss_ref[...] = _dft_sq_sum(attrib, idx_r, tif, use_sin=False)
    iss_ref[...] = _dft_sq_sum(attrib, idx_i, tif, use_sin=True)
    v4r_ref[...] = v4r
    v4i_ref[...] = v4i
    idxi_ref[...] = jnp.concatenate(idx_i, axis=1)


def _pass2_kernel(rows_ref, x_ref, idx_ref, s_ref, out_ref):
    del rows_ref
    attrib = x_ref[...] * s_ref[...]
    ti = jax.lax.broadcasted_iota(jnp.int32, (_C, _T), 1)
    idx = idx_ref[...]
    ss = jnp.zeros((_C, 1), jnp.float32)
    for k in range(_K):
        prod = jnp.bitwise_and(idx[:, k:k + 1] * ti, _T - 1)
        ang = prod.astype(jnp.float32) * _TWO_PI_OVER_T
        b = jnp.sum(attrib * jnp.sin(ang), axis=1, keepdims=True)
        ss = ss + b * b
    out_ref[...] = jnp.full((1, 1, 128), jnp.sum(ss), jnp.float32)


def kernel(input, predictions, expl_p_real, expl_p_imag, W):
    del predictions
    s = jnp.sum(W, axis=1).reshape(1, _T)
    x2 = input.reshape(_B * _C, _T)
    er2 = expl_p_real.reshape(_B * _C, _T)
    ei2 = expl_p_imag.reshape(_B * _C, _T)

    nb = _B // _BB
    idxi, v4r, v4i, rss, iss_self = pl.pallas_call(
        _pass1_kernel,
        grid=(nb,),
        compiler_params=pltpu.CompilerParams(
            dimension_semantics=("parallel",)),
        in_specs=[
            pl.BlockSpec((_R, _T), lambda i: (i, 0)),
            pl.BlockSpec((_R, _T), lambda i: (i, 0)),
            pl.BlockSpec((_R, _T), lambda i: (i, 0)),
            pl.BlockSpec((1, _T), lambda i: (0, 0)),
        ],
        out_specs=[
            pl.BlockSpec((_R, _K), lambda i: (i, 0)),
            pl.BlockSpec((_R, 1), lambda i: (i, 0)),
            pl.BlockSpec((_R, 1), lambda i: (i, 0)),
            pl.BlockSpec((_R, 1), lambda i: (i, 0)),
            pl.BlockSpec((_R, 1), lambda i: (i, 0)),
        ],
        out_shape=[
            jax.ShapeDtypeStruct((_B * _C, _K), jnp.int32),
            jax.ShapeDtypeStruct((_B * _C, 1), jnp.float32),
            jax.ShapeDtypeStruct((_B * _C, 1), jnp.float32),
            jax.ShapeDtypeStruct((_B * _C, 1), jnp.float32),
            jax.ShapeDtypeStruct((_B * _C, 1), jnp.float32),
        ],
    )(x2, er2, ei2, s)

    real_invalid = v4r.reshape(_B, _C).min(axis=1) < _THRESHOLD
    imag_invalid = v4i.reshape(_B, _C).min(axis=1) < _THRESHOLD
    kept_real = jnp.logical_not(real_invalid)
    kept_imag = jnp.logical_not(imag_invalid)
    real_count = jnp.sum(kept_real)
    imag_count = jnp.sum(kept_imag)
    ck = float(_C * _K)

    rtot = jnp.sum(rss.reshape(_B, _C).sum(axis=1)
                   * kept_real.astype(jnp.float32))
    rden = jnp.maximum(real_count.astype(jnp.float32), 1.0)
    real_loss = jnp.where(real_count > 0, rtot / (rden * ck) / rden,
                          jnp.float32(0.0))

    def _imag_common(_):
        return jnp.sum(iss_self)

    def _imag_paired(_):
        row_ids = jnp.arange(_B)
        fft_mask = jnp.where(real_count > 0, kept_real,
                             jnp.ones_like(kept_real))
        sym_mask = jnp.logical_xor(fft_mask, imag_invalid)
        sym_sorted = jnp.sort(jnp.where(sym_mask, row_ids, _B))
        kept_sorted = jnp.sort(jnp.where(kept_imag, row_ids, _B))
        rows_fft = jnp.minimum(sym_sorted, _B - 1).astype(jnp.int32)
        rows_idx = jnp.minimum(kept_sorted, _B - 1).astype(jnp.int32)
        idx_pair = idxi.reshape(_B, _C, _K)[rows_idx].reshape(_B * _C, _K)
        pair_valid = (row_ids < imag_count).astype(jnp.float32)
        iss2 = pl.pallas_call(
            _pass2_kernel,
            grid_spec=pltpu.PrefetchScalarGridSpec(
                num_scalar_prefetch=1,
                grid=(_B,),
                in_specs=[
                    pl.BlockSpec((_C, _T), lambda r, rows: (rows[r], 0)),
                    pl.BlockSpec((_C, _K), lambda r, rows: (r, 0)),
                    pl.BlockSpec((1, _T), lambda r, rows: (0, 0)),
                ],
                out_specs=pl.BlockSpec((1, 1, 128), lambda r, rows: (r, 0, 0)),
            ),
            out_shape=jax.ShapeDtypeStruct((_B, 1, 128), jnp.float32),
        )(rows_fft, x2, idx_pair, s)
        return jnp.sum(iss2[:, 0, 0] * pair_valid)

    all_valid = jnp.logical_and(real_count == _B, imag_count == _B)
    itot = jax.lax.cond(all_valid, _imag_common, _imag_paired, None)
    iden = jnp.maximum(imag_count.astype(jnp.float32), 1.0)
    imag_loss = jnp.where(imag_count > 0, itot / (iden * ck) / iden,
                          jnp.float32(0.0))
    return real_loss + imag_loss
